# baseline (speedup 1.0000x reference)
import jax
import jax.numpy as jnp
from jax.experimental import pallas as pl

_LAMBDA2 = 1.0
_K = 1024
_D = 64
_BM = 2304


def _cluster_kernel(z_ref, c_ref, rev_ref, q_ref, idx_ref, loss_ref):
    i = pl.program_id(0)
    z = z_ref[...]
    c = c_ref[...]
    z2 = jnp.sum(z * z, axis=1, keepdims=True)
    c2 = jnp.sum(c * c, axis=1).reshape(1, _K)
    zc = jax.lax.dot_general(
        z, c, (((1,), (1,)), ((), ())),
        preferred_element_type=jnp.float32)
    dist = (z2 + c2) - 2.0 * zc
    minv = jnp.min(dist, axis=1, keepdims=True)
    w = jnp.where(dist == minv, rev_ref[...], 0.0)
    wmax = jnp.max(w, axis=1, keepdims=True)
    idx_ref[...] = (float(_K) - wmax).astype(jnp.int32)
    onehot = (w == wmax).astype(jnp.float32)
    q_ref[...] = jax.lax.dot_general(
        onehot, c, (((1,), (0,)), ((), ())),
        preferred_element_type=jnp.float32)
    part = jnp.reshape(_LAMBDA2 * 0.5 * jnp.sum(minv), (1, 1))

    @pl.when(i == 0)
    def _():
        loss_ref[...] = jnp.zeros((1, 1), jnp.float32)

    loss_ref[...] += part


def kernel(z, clusters):
    B, N, D = z.shape
    M = B * N
    nb = M // _BM
    zf = z.reshape(M, D)
    rev = (float(_K) - jnp.arange(_K, dtype=jnp.float32))[None, :]

    q, idx_col, loss = pl.pallas_call(
        _cluster_kernel,
        grid=(nb,),
        in_specs=[
            pl.BlockSpec((_BM, D), lambda i: (i, 0)),
            pl.BlockSpec((_K, D), lambda i: (0, 0)),
            pl.BlockSpec((1, _K), lambda i: (0, 0)),
        ],
        out_specs=[
            pl.BlockSpec((_BM, D), lambda i: (i, 0)),
            pl.BlockSpec((_BM, 1), lambda i: (i, 0)),
            pl.BlockSpec((1, 1), lambda i: (0, 0)),
        ],
        out_shape=[
            jax.ShapeDtypeStruct((M, D), jnp.float32),
            jax.ShapeDtypeStruct((M, 1), jnp.int32),
            jax.ShapeDtypeStruct((1, 1), jnp.float32),
        ],
    )(zf, clusters, rev)

    return q.reshape(B, N, D), idx_col.reshape(B, N), loss.reshape(())
